# baseline (speedup 1.0000x reference)
import jax
import jax.numpy as jnp
from jax import lax
from jax.experimental import pallas as pl
from jax.experimental.pallas import tpu as pltpu
from jax.experimental.pallas import tpu_sc as plsc

_NC = 2
_NS = 16
_NW = _NC * _NS
_L = 16
_BINS = 256
_INV_W = 256.0 / 255.0

_ROWS = 8192
_COLS = 4096
_ROWS_PER_W = _ROWS // _NW
_CROWS = 8
_NBUF = 2
_NCHUNK = _ROWS_PER_W // _CROWS
_UNROLL = 8


def _sc_body(x_hbm, out_hbm, buf, hist2d, histv, sem0, sem1):
    sems = (sem0, sem1)
    wid = lax.axis_index("s") * _NC + lax.axis_index("c")
    base = wid * _ROWS_PER_W

    lanes = lax.iota(jnp.int32, _L)
    lanes16 = lanes * _L
    ones = jnp.full((_L,), 1.0, jnp.float32)
    zeros = jnp.zeros((_L,), jnp.float32)

    for r in range(_BINS):
        hist2d[pl.ds(r * _L, _L)] = zeros

    for b in range(_NBUF):
        pltpu.async_copy(
            x_hbm.at[pl.ds(base + b * _CROWS, _CROWS)], buf.at[b], sems[b])

    def chunk_pair(j, carry):
        for b in range(_NBUF):
            c = j * _NBUF + b
            src = x_hbm.at[pl.ds(base + c * _CROWS, _CROWS)]
            pltpu.make_async_copy(src, buf.at[b], sems[b]).wait()

            for row in range(_CROWS):

                @plsc.parallel_loop(0, _COLS, _L, unroll=_UNROLL)
                def vec_body(off):
                    x = buf[b, row, pl.ds(off, _L)]
                    y = jnp.minimum(x * (16.0 * _INV_W), 4095.999755859375)
                    flat = y.astype(jnp.int32)
                    plsc.addupdate_scatter(hist2d, [flat], ones)

            nxt = c + _NBUF

            @pl.when(nxt < _NCHUNK)
            def _():
                pltpu.async_copy(
                    x_hbm.at[pl.ds(base + nxt * _CROWS, _CROWS)],
                    buf.at[b], sems[b])
        return carry

    lax.fori_loop(0, _NCHUNK // _NBUF, chunk_pair, 0)

    for g in range(_BINS // _L):
        acc = zeros
        for r in range(_L):
            addr = lanes16 + (g * _L * _L + r)
            acc = acc + plsc.load_gather(hist2d, [addr])
        histv[pl.ds(g * _L, _L)] = acc

    pltpu.sync_copy(histv, out_hbm.at[pl.ds(wid * _BINS, _BINS)])


_sc_hist = pl.kernel(
    _sc_body,
    out_type=jax.ShapeDtypeStruct((_NW * _BINS,), jnp.float32),
    mesh=plsc.VectorSubcoreMesh(core_axis_name="c", subcore_axis_name="s"),
    compiler_params=pltpu.CompilerParams(needs_layout_passes=False),
    scratch_types=[
        pltpu.VMEM((_NBUF, _CROWS, _COLS), jnp.float32),
        pltpu.VMEM((_BINS * _L,), jnp.float32),
        pltpu.VMEM((_BINS,), jnp.float32),
        pltpu.SemaphoreType.DMA,
        pltpu.SemaphoreType.DMA,
    ],
)


def _tc_reduce(parts_ref, bs_ref, hist_ref, count_ref):
    p = parts_ref[...]
    h = jnp.sum(p, axis=0, keepdims=True)
    hist_ref[...] = h
    col = lax.broadcasted_iota(jnp.int32, (1, _BINS), 1)
    h0 = jnp.sum(jnp.where(col == 0, h, 0.0))
    count_ref[...] = jnp.zeros((1, _BINS), jnp.float32) + bs_ref[0, 0] * h0


def kernel(batchsize, input):
    parts = _sc_hist(input).reshape(_NW, _BINS)
    bs = jnp.asarray(batchsize, jnp.float32).reshape(1, 1)
    hist, count = pl.pallas_call(
        _tc_reduce,
        out_shape=(
            jax.ShapeDtypeStruct((1, _BINS), jnp.float32),
            jax.ShapeDtypeStruct((1, _BINS), jnp.float32),
        ),
    )(parts, bs)
    return hist.reshape(_BINS), count.reshape(_BINS)


# speedup vs baseline: 1.3591x; 1.3591x over previous
import jax
import jax.numpy as jnp
import numpy as np
from jax import lax
from jax.experimental import pallas as pl
from jax.experimental.pallas import tpu as pltpu
from jax.experimental.pallas import tpu_sc as plsc

_NC = 2
_NS = 16
_NW = _NC * _NS
_L = 16
_BINS = 256
_INV_W = 256.0 / 255.0
_INV_W_DN = float(np.nextafter(np.float32(_INV_W), np.float32(0.0)))

_ROWS = 8192
_COLS = 4096
_ROWS_PER_W = _ROWS // _NW
_CROWS = 8
_NBUF = 2
_NCHUNK = _ROWS_PER_W // _CROWS
_UNROLL = 8


def _sc_body(x_hbm, out_hbm, buf, hist2d, histv, sem0, sem1):
    sems = (sem0, sem1)
    wid = lax.axis_index("s") * _NC + lax.axis_index("c")
    base = wid * _ROWS_PER_W

    lanes = lax.iota(jnp.int32, _L)
    lanes16 = lanes * _L
    ones = jnp.full((_L,), 1.0, jnp.float32)
    zeros = jnp.zeros((_L,), jnp.float32)

    for r in range(_BINS):
        hist2d[pl.ds(r * _L, _L)] = zeros

    for b in range(_NBUF):
        pltpu.async_copy(
            x_hbm.at[pl.ds(base + b * _CROWS, _CROWS)], buf.at[b], sems[b])

    def chunk_pair(j, carry):
        for b in range(_NBUF):
            c = j * _NBUF + b
            src = x_hbm.at[pl.ds(base + c * _CROWS, _CROWS)]
            pltpu.make_async_copy(src, buf.at[b], sems[b]).wait()

            for row in range(_CROWS):

                @plsc.parallel_loop(0, _COLS, _L, unroll=_UNROLL)
                def vec_body(off):
                    x = buf[b, row, pl.ds(off, _L)]
                    idx = (x * _INV_W_DN).astype(jnp.int32)
                    flat = lax.shift_left(idx, 4) + lanes
                    plsc.addupdate_scatter(hist2d, [flat], ones)

            nxt = c + _NBUF

            @pl.when(nxt < _NCHUNK)
            def _():
                pltpu.async_copy(
                    x_hbm.at[pl.ds(base + nxt * _CROWS, _CROWS)],
                    buf.at[b], sems[b])
        return carry

    lax.fori_loop(0, _NCHUNK // _NBUF, chunk_pair, 0)

    for g in range(_BINS // _L):
        acc = zeros
        for r in range(_L):
            addr = lanes16 + (g * _L * _L + r)
            acc = acc + plsc.load_gather(hist2d, [addr])
        histv[pl.ds(g * _L, _L)] = acc

    pltpu.sync_copy(histv, out_hbm.at[pl.ds(wid * _BINS, _BINS)])


_sc_hist = pl.kernel(
    _sc_body,
    out_type=jax.ShapeDtypeStruct((_NW * _BINS,), jnp.float32),
    mesh=plsc.VectorSubcoreMesh(core_axis_name="c", subcore_axis_name="s"),
    compiler_params=pltpu.CompilerParams(needs_layout_passes=False),
    scratch_types=[
        pltpu.VMEM((_NBUF, _CROWS, _COLS), jnp.float32),
        pltpu.VMEM((_BINS * _L,), jnp.float32),
        pltpu.VMEM((_BINS,), jnp.float32),
        pltpu.SemaphoreType.DMA,
        pltpu.SemaphoreType.DMA,
    ],
)


def _tc_reduce(parts_ref, bs_ref, hist_ref, count_ref):
    p = parts_ref[...]
    h = jnp.sum(p, axis=0, keepdims=True)
    hist_ref[...] = h
    col = lax.broadcasted_iota(jnp.int32, (1, _BINS), 1)
    h0 = jnp.sum(jnp.where(col == 0, h, 0.0))
    count_ref[...] = jnp.zeros((1, _BINS), jnp.float32) + bs_ref[0, 0] * h0


def kernel(batchsize, input):
    parts = _sc_hist(input).reshape(_NW, _BINS)
    bs = jnp.asarray(batchsize, jnp.float32).reshape(1, 1)
    hist, count = pl.pallas_call(
        _tc_reduce,
        out_shape=(
            jax.ShapeDtypeStruct((1, _BINS), jnp.float32),
            jax.ShapeDtypeStruct((1, _BINS), jnp.float32),
        ),
    )(parts, bs)
    return hist.reshape(_BINS), count.reshape(_BINS)
